# baseline (speedup 1.0000x reference)
import functools

import jax
import jax.numpy as jnp
from jax import lax
from jax.experimental import pallas as pl
from jax.experimental.pallas import tpu as pltpu
from jax.experimental.pallas import tpu_sc as plsc

NUM_ROWS = 1000000
DIM = 64
BATCH = 4096
SEQ = 200
NC, NS = 2, 16
NW = NC * NS
ROWS_PER_W = BATCH // NW
NBUF = 8
G = 6
S = NBUF - G
N_GROUPS = ROWS_PER_W // NBUF

_mesh = plsc.VectorSubcoreMesh(core_axis_name="c", subcore_axis_name="s")


@functools.partial(
    pl.kernel,
    mesh=_mesh,
    out_type=jax.ShapeDtypeStruct((BATCH, SEQ, DIM), jnp.float32),
    scratch_types=[
        pltpu.VMEM((ROWS_PER_W, SEQ), jnp.int32),
        pltpu.VMEM((NBUF, SEQ, DIM), jnp.float32),
        pltpu.SemaphoreType.DMA,
        pltpu.SemaphoreType.DMA,
    ],
    compiler_params=pltpu.CompilerParams(use_tc_tiling_on_sc=False),
)
def _gather_kernel(idx_hbm, table_hbm, out_hbm, idx_v, rows_v, gsem, ssem):
    wid = lax.axis_index("s") * NC + lax.axis_index("c")
    base = wid * ROWS_PER_W
    pltpu.sync_copy(idx_hbm.at[pl.ds(base, ROWS_PER_W)], idx_v)

    def gather(row, buf):
        pltpu.async_copy(table_hbm.at[idx_v.at[row]], rows_v.at[buf], gsem)

    def store(row, buf):
        pltpu.async_copy(rows_v.at[buf], out_hbm.at[base + row], ssem)

    def wait_gather(buf):
        pltpu.make_async_copy(out_hbm.at[base], rows_v.at[buf], gsem).wait()

    def wait_store(buf):
        pltpu.make_async_copy(rows_v.at[buf], out_hbm.at[base], ssem).wait()

    for b in range(G):
        gather(b, b)

    for b in range(NBUF):
        wait_gather(b)
        store(b, b)
        if b >= S:
            wait_store(b)
        gather(b + G, (b + G) % NBUF)

    def group(g, carry):
        j0 = g * NBUF
        for b in range(NBUF):
            j = j0 + b
            wait_gather(b)
            store(j, b)
            wait_store(b)
            gather(j + G, (b + G) % NBUF)
        return carry

    lax.fori_loop(1, N_GROUPS - 1, group, 0)

    j0 = (N_GROUPS - 1) * NBUF
    for b in range(NBUF):
        j = j0 + b
        wait_gather(b)
        store(j, b)
        wait_store(b)
        if j + G < ROWS_PER_W:
            gather(j + G, (b + G) % NBUF)

    for b in range(S):
        wait_store(b)


def kernel(input_ids, weight):
    return _gather_kernel(input_ids.astype(jnp.int32), weight)
